# baseline (speedup 1.0000x reference)
import functools

import jax
import jax.numpy as jnp
from jax import lax
from jax.experimental import pallas as pl
from jax.experimental.pallas import tpu as pltpu
from jax.experimental.pallas import tpu_sc as plsc

NUM_ENTITY = 10000
NUM_RELATION = 200
NUM_FACT = 320000
DIM = 128

NC = 2
NS = 16
NW = NC * NS
L = 16

FACTS_PER_W = NUM_FACT // NW
BLK = 80
SPL = 64
CHUNK = 2000
BLKS_PER_CHUNK = CHUNK // BLK
PAIRS = (BLKS_PER_CHUNK - 1) // 2
CHUNKS = FACTS_PER_W // CHUNK
ROWS_PER_TILE = 624
REM_ROWS = NUM_ENTITY - NS * ROWS_PER_TILE

_mesh = plsc.VectorSubcoreMesh(
    core_axis_name="c", subcore_axis_name="s", num_cores=NC, num_subcores=NS)


@functools.partial(
    pl.kernel,
    out_type=jax.ShapeDtypeStruct((NC * NUM_ENTITY, DIM), jnp.float32),
    mesh=_mesh,
    scratch_types=dict(
        hd_st=pltpu.VMEM((CHUNK,), jnp.int32),
        rl_st=pltpu.VMEM((CHUNK,), jnp.int32),
        bi_st=pltpu.VMEM((CHUNK,), jnp.int32),
        tl_st=pltpu.VMEM((CHUNK,), jnp.int32),
        ridx=[pltpu.VMEM((BLK,), jnp.int32) for _ in range(2)],
        tidxa=[pltpu.VMEM((SPL,), jnp.int32) for _ in range(2)],
        tidxb=[pltpu.VMEM((BLK - SPL,), jnp.int32) for _ in range(2)],
        hbuf=[pltpu.VMEM((BLK, DIM), jnp.float32) for _ in range(2)],
        pbuf=[pltpu.VMEM((BLK, DIM), jnp.float32) for _ in range(2)],
        sem_st=pltpu.SemaphoreType.DMA,
        sem_h=[pltpu.SemaphoreType.DMA for _ in range(2)],
        sem_r=[pltpu.SemaphoreType.DMA for _ in range(2)],
        sem_s=[pltpu.SemaphoreType.DMA for _ in range(2)],
        accum=pltpu.VMEM_SHARED((NUM_ENTITY, DIM), jnp.float32),
    ),
    compiler_params=pltpu.CompilerParams(use_tc_tiling_on_sc=False),
)
def _sc_message_pass(entity_hbm, rel_hbm, heads_hbm, rels_hbm, ids_hbm,
                     tails_hbm, part_hbm, hd_st, rl_st, bi_st, tl_st, ridx,
                     tidxa, tidxb, hbuf, pbuf, sem_st, sem_h, sem_r, sem_s,
                     accum):
  core = lax.axis_index("c")
  sid = lax.axis_index("s")
  w = core * NS + sid

  zero = jnp.zeros((L,), jnp.float32)

  def _zrow(r, _):
    for j in range(DIM // L):
      hbuf[0][r, pl.ds(j * L, L)] = zero
    return 0
  lax.fori_loop(0, BLK, _zrow, 0)
  for k in range(7):
    pltpu.sync_copy(hbuf[0],
                    accum.at[pl.ds(sid * ROWS_PER_TILE + k * BLK, BLK)])
  pltpu.sync_copy(hbuf[0].at[pl.ds(0, 64)],
                  accum.at[pl.ds(sid * ROWS_PER_TILE + 7 * BLK, 64)])
  @pl.when(sid == NS - 1)
  def _zero_tail():
    pltpu.sync_copy(hbuf[0].at[pl.ds(0, REM_ROWS)],
                    accum.at[pl.ds(NS * ROWS_PER_TILE, REM_ROWS)])
  plsc.subcore_barrier()

  def _idx(off, p):
    for j in range(BLK // L):
      src = pl.ds(off + j * L, L)
      ridx[p][pl.ds(j * L, L)] = rl_st[src] + bi_st[src] * NUM_RELATION
    for j in range(SPL // L):
      tidxa[p][pl.ds(j * L, L)] = tl_st[pl.ds(off + j * L, L)]
    for j in range((BLK - SPL) // L):
      tidxb[p][pl.ds(j * L, L)] = tl_st[pl.ds(off + SPL + j * L, L)]

  def _issue_gathers(off, p):
    pltpu.async_copy(entity_hbm.at[hd_st.at[pl.ds(off, BLK)]], hbuf[p],
                     sem_h[p])
    pltpu.async_copy(rel_hbm.at[ridx[p]], pbuf[p], sem_r[p])

  def _wait_gathers(off, p):
    pltpu.make_async_copy(entity_hbm.at[hd_st.at[pl.ds(off, BLK)]], hbuf[p],
                          sem_h[p]).wait()
    pltpu.make_async_copy(rel_hbm.at[ridx[p]], pbuf[p], sem_r[p]).wait()

  def _mul_rows(p, lo, hi):
    hb, pb = hbuf[p], pbuf[p]
    def _mrow(i, _):
      r = 2 * i
      for rr in (r, r + 1):
        for j in range(DIM // L):
          s = pl.ds(j * L, L)
          pb[rr, s] = hb[rr, s] * pb[rr, s]
      return 0
    lax.fori_loop(lo // 2, hi // 2, _mrow, 0)

  def _mul_scatter(p):
    _mul_rows(p, 0, SPL)
    pltpu.async_copy(pbuf[p].at[pl.ds(0, SPL)], accum.at[tidxa[p]], sem_s[p],
                     add=True)
    _mul_rows(p, SPL, BLK)
    pltpu.sync_copy(pbuf[p].at[pl.ds(SPL, BLK - SPL)], accum.at[tidxb[p]],
                    add=True)
    pltpu.make_async_copy(pbuf[p].at[pl.ds(0, SPL)], accum.at[tidxa[p]],
                          sem_s[p]).wait()

  def _chunk(c, _):
    base = w * FACTS_PER_W + c * CHUNK
    cps = [
        pltpu.async_copy(heads_hbm.at[pl.ds(base, CHUNK)], hd_st, sem_st),
        pltpu.async_copy(rels_hbm.at[pl.ds(base, CHUNK)], rl_st, sem_st),
        pltpu.async_copy(ids_hbm.at[pl.ds(base, CHUNK)], bi_st, sem_st),
        pltpu.async_copy(tails_hbm.at[pl.ds(base, CHUNK)], tl_st, sem_st),
    ]
    for cp in cps:
      cp.wait()

    _idx(0, 0)
    _issue_gathers(0, 0)

    def _pair(i, _):
      b1 = 2 * i + 1
      _idx(b1 * BLK, 1)
      _issue_gathers(b1 * BLK, 1)
      _wait_gathers((b1 - 1) * BLK, 0)
      _mul_scatter(0)

      b2 = 2 * i + 2
      _idx(b2 * BLK, 0)
      _issue_gathers(b2 * BLK, 0)
      _wait_gathers((b2 - 1) * BLK, 1)
      _mul_scatter(1)
      return 0

    lax.fori_loop(0, PAIRS, _pair, 0)

    _wait_gathers((BLKS_PER_CHUNK - 1) * BLK, 0)
    _mul_scatter(0)
    return 0

  lax.fori_loop(0, CHUNKS, _chunk, 0)

  plsc.subcore_barrier()
  for k in range(7):
    r0 = sid * ROWS_PER_TILE + k * BLK
    pltpu.sync_copy(accum.at[pl.ds(r0, BLK)], hbuf[0])
    pltpu.sync_copy(hbuf[0], part_hbm.at[pl.ds(core * NUM_ENTITY + r0, BLK)])
  r0 = sid * ROWS_PER_TILE + 7 * BLK
  pltpu.sync_copy(accum.at[pl.ds(r0, 64)], hbuf[0].at[pl.ds(0, 64)])
  pltpu.sync_copy(hbuf[0].at[pl.ds(0, 64)],
                  part_hbm.at[pl.ds(core * NUM_ENTITY + r0, 64)])
  @pl.when(sid == NS - 1)
  def _drain_tail():
    r1 = NS * ROWS_PER_TILE
    pltpu.sync_copy(accum.at[pl.ds(r1, REM_ROWS)],
                    hbuf[1].at[pl.ds(0, REM_ROWS)])
    pltpu.sync_copy(hbuf[1].at[pl.ds(0, REM_ROWS)],
                    part_hbm.at[pl.ds(core * NUM_ENTITY + r1, REM_ROWS)])


def _combine_body(a_ref, b_ref, o_ref):
  o_ref[...] = a_ref[...] + b_ref[...]


_combine = pl.pallas_call(
    _combine_body,
    grid=(10,),
    in_specs=[
        pl.BlockSpec((NUM_ENTITY // 10, DIM), lambda i: (i, 0)),
        pl.BlockSpec((NUM_ENTITY // 10, DIM), lambda i: (i + 10, 0)),
    ],
    out_specs=pl.BlockSpec((NUM_ENTITY // 10, DIM), lambda i: (i, 0)),
    out_shape=jax.ShapeDtypeStruct((NUM_ENTITY, DIM), jnp.float32),
)


def kernel(local_entity_emb, rel_emb, batch_heads, batch_rels, batch_tails,
           batch_ids):
  part = _sc_message_pass(local_entity_emb, rel_emb, batch_heads, batch_rels,
                          batch_ids, batch_tails)
  return _combine(part, part)


# speedup vs baseline: 1.0307x; 1.0307x over previous
import functools

import jax
import jax.numpy as jnp
from jax import lax
from jax.experimental import pallas as pl
from jax.experimental.pallas import tpu as pltpu
from jax.experimental.pallas import tpu_sc as plsc

NUM_ENTITY = 10000
NUM_RELATION = 200
NUM_FACT = 320000
DIM = 128

NC = 2
NS = 16
NW = NC * NS
L = 16

FACTS_PER_W = NUM_FACT // NW
BLK = 80
SPL = 48
CHUNK = 2000
BLKS_PER_CHUNK = CHUNK // BLK
PAIRS = (BLKS_PER_CHUNK - 1) // 2
CHUNKS = FACTS_PER_W // CHUNK
ROWS_PER_TILE = 624
REM_ROWS = NUM_ENTITY - NS * ROWS_PER_TILE

_mesh = plsc.VectorSubcoreMesh(
    core_axis_name="c", subcore_axis_name="s", num_cores=NC, num_subcores=NS)


@functools.partial(
    pl.kernel,
    out_type=jax.ShapeDtypeStruct((NC * NUM_ENTITY, DIM), jnp.float32),
    mesh=_mesh,
    scratch_types=dict(
        hd_st=pltpu.VMEM((CHUNK,), jnp.int32),
        rl_st=pltpu.VMEM((CHUNK,), jnp.int32),
        bi_st=pltpu.VMEM((CHUNK,), jnp.int32),
        tl_st=pltpu.VMEM((CHUNK,), jnp.int32),
        ridx=[pltpu.VMEM((BLK,), jnp.int32) for _ in range(2)],
        tidxa=[pltpu.VMEM((SPL,), jnp.int32) for _ in range(2)],
        tidxb=[pltpu.VMEM((BLK - SPL,), jnp.int32) for _ in range(2)],
        hbuf=[pltpu.VMEM((BLK, DIM), jnp.float32) for _ in range(2)],
        pbuf=[pltpu.VMEM((BLK, DIM), jnp.float32) for _ in range(2)],
        sem_st=pltpu.SemaphoreType.DMA,
        sem_h=[pltpu.SemaphoreType.DMA for _ in range(2)],
        sem_r=[pltpu.SemaphoreType.DMA for _ in range(2)],
        sem_s=[pltpu.SemaphoreType.DMA for _ in range(2)],
        accum=pltpu.VMEM_SHARED((NUM_ENTITY, DIM), jnp.float32),
    ),
    compiler_params=pltpu.CompilerParams(use_tc_tiling_on_sc=False),
)
def _sc_message_pass(entity_hbm, rel_hbm, heads_hbm, rels_hbm, ids_hbm,
                     tails_hbm, part_hbm, hd_st, rl_st, bi_st, tl_st, ridx,
                     tidxa, tidxb, hbuf, pbuf, sem_st, sem_h, sem_r, sem_s,
                     accum):
  core = lax.axis_index("c")
  sid = lax.axis_index("s")
  w = core * NS + sid

  zero = jnp.zeros((L,), jnp.float32)

  def _zrow(r, _):
    for j in range(DIM // L):
      hbuf[0][r, pl.ds(j * L, L)] = zero
    return 0
  lax.fori_loop(0, BLK, _zrow, 0)
  for k in range(7):
    pltpu.sync_copy(hbuf[0],
                    accum.at[pl.ds(sid * ROWS_PER_TILE + k * BLK, BLK)])
  pltpu.sync_copy(hbuf[0].at[pl.ds(0, 64)],
                  accum.at[pl.ds(sid * ROWS_PER_TILE + 7 * BLK, 64)])
  @pl.when(sid == NS - 1)
  def _zero_tail():
    pltpu.sync_copy(hbuf[0].at[pl.ds(0, REM_ROWS)],
                    accum.at[pl.ds(NS * ROWS_PER_TILE, REM_ROWS)])
  plsc.subcore_barrier()

  def _idx(off, p):
    for j in range(BLK // L):
      src = pl.ds(off + j * L, L)
      ridx[p][pl.ds(j * L, L)] = rl_st[src] + bi_st[src] * NUM_RELATION
    for j in range(SPL // L):
      tidxa[p][pl.ds(j * L, L)] = tl_st[pl.ds(off + j * L, L)]
    for j in range((BLK - SPL) // L):
      tidxb[p][pl.ds(j * L, L)] = tl_st[pl.ds(off + SPL + j * L, L)]

  def _issue_gathers(off, p):
    pltpu.async_copy(entity_hbm.at[hd_st.at[pl.ds(off, BLK)]], hbuf[p],
                     sem_h[p])
    pltpu.async_copy(rel_hbm.at[ridx[p]], pbuf[p], sem_r[p])

  def _wait_gathers(off, p):
    pltpu.make_async_copy(entity_hbm.at[hd_st.at[pl.ds(off, BLK)]], hbuf[p],
                          sem_h[p]).wait()
    pltpu.make_async_copy(rel_hbm.at[ridx[p]], pbuf[p], sem_r[p]).wait()

  def _mul_rows(p, lo, hi):
    hb, pb = hbuf[p], pbuf[p]
    def _mrow(i, _):
      r = 2 * i
      for rr in (r, r + 1):
        for j in range(DIM // L):
          s = pl.ds(j * L, L)
          pb[rr, s] = hb[rr, s] * pb[rr, s]
      return 0
    lax.fori_loop(lo // 2, hi // 2, _mrow, 0)

  def _mul_scatter(p):
    _mul_rows(p, 0, SPL)
    pltpu.async_copy(pbuf[p].at[pl.ds(0, SPL)], accum.at[tidxa[p]], sem_s[p],
                     add=True)
    _mul_rows(p, SPL, BLK)
    pltpu.sync_copy(pbuf[p].at[pl.ds(SPL, BLK - SPL)], accum.at[tidxb[p]],
                    add=True)
    pltpu.make_async_copy(pbuf[p].at[pl.ds(0, SPL)], accum.at[tidxa[p]],
                          sem_s[p]).wait()

  def _stage_issue(base):
    pltpu.async_copy(heads_hbm.at[pl.ds(base, CHUNK)], hd_st, sem_st)
    pltpu.async_copy(rels_hbm.at[pl.ds(base, CHUNK)], rl_st, sem_st)
    pltpu.async_copy(ids_hbm.at[pl.ds(base, CHUNK)], bi_st, sem_st)
    pltpu.async_copy(tails_hbm.at[pl.ds(base, CHUNK)], tl_st, sem_st)

  def _stage_wait(base):
    for src, dst in ((heads_hbm, hd_st), (rels_hbm, rl_st), (ids_hbm, bi_st),
                     (tails_hbm, tl_st)):
      pltpu.make_async_copy(src.at[pl.ds(base, CHUNK)], dst, sem_st).wait()

  def _chunk(c, _):
    base = w * FACTS_PER_W + c * CHUNK
    _stage_wait(base)

    _idx(0, 0)
    _issue_gathers(0, 0)

    def _pair(i, _):
      b1 = 2 * i + 1
      _idx(b1 * BLK, 1)
      _issue_gathers(b1 * BLK, 1)
      _wait_gathers((b1 - 1) * BLK, 0)
      _mul_scatter(0)

      b2 = 2 * i + 2
      _idx(b2 * BLK, 0)
      _issue_gathers(b2 * BLK, 0)
      _wait_gathers((b2 - 1) * BLK, 1)
      _mul_scatter(1)
      return 0

    lax.fori_loop(0, PAIRS, _pair, 0)

    _wait_gathers((BLKS_PER_CHUNK - 1) * BLK, 0)
    @pl.when(c < CHUNKS - 1)
    def _prefetch():
      _stage_issue(base + CHUNK)
    _mul_scatter(0)
    return 0

  _stage_issue(w * FACTS_PER_W)
  lax.fori_loop(0, CHUNKS, _chunk, 0)

  plsc.subcore_barrier()
  for k in range(7):
    r0 = sid * ROWS_PER_TILE + k * BLK
    pltpu.sync_copy(accum.at[pl.ds(r0, BLK)], hbuf[0])
    pltpu.sync_copy(hbuf[0], part_hbm.at[pl.ds(core * NUM_ENTITY + r0, BLK)])
  r0 = sid * ROWS_PER_TILE + 7 * BLK
  pltpu.sync_copy(accum.at[pl.ds(r0, 64)], hbuf[0].at[pl.ds(0, 64)])
  pltpu.sync_copy(hbuf[0].at[pl.ds(0, 64)],
                  part_hbm.at[pl.ds(core * NUM_ENTITY + r0, 64)])
  @pl.when(sid == NS - 1)
  def _drain_tail():
    r1 = NS * ROWS_PER_TILE
    pltpu.sync_copy(accum.at[pl.ds(r1, REM_ROWS)],
                    hbuf[1].at[pl.ds(0, REM_ROWS)])
    pltpu.sync_copy(hbuf[1].at[pl.ds(0, REM_ROWS)],
                    part_hbm.at[pl.ds(core * NUM_ENTITY + r1, REM_ROWS)])


def _combine_body(a_ref, b_ref, o_ref):
  o_ref[...] = a_ref[...] + b_ref[...]


_combine = pl.pallas_call(
    _combine_body,
    grid=(10,),
    in_specs=[
        pl.BlockSpec((NUM_ENTITY // 10, DIM), lambda i: (i, 0)),
        pl.BlockSpec((NUM_ENTITY // 10, DIM), lambda i: (i + 10, 0)),
    ],
    out_specs=pl.BlockSpec((NUM_ENTITY // 10, DIM), lambda i: (i, 0)),
    out_shape=jax.ShapeDtypeStruct((NUM_ENTITY, DIM), jnp.float32),
)


def kernel(local_entity_emb, rel_emb, batch_heads, batch_rels, batch_tails,
           batch_ids):
  part = _sc_message_pass(local_entity_emb, rel_emb, batch_heads, batch_rels,
                          batch_ids, batch_tails)
  return _combine(part, part)
